# baseline (speedup 1.0000x reference)
import functools

import jax
import jax.numpy as jnp
from jax import lax
from jax.experimental import pallas as pl
from jax.experimental.pallas import tpu as pltpu
from jax.experimental.pallas import tpu_sc as plsc

VOCAB = 100000
EMBED_DIM = 128
TOKENS_SHAPE = (4096, 50)
B = TOKENS_SHAPE[0] * TOKENS_SHAPE[1]

NUM_CORES = 2
NUM_SUBCORES = 16
NW = NUM_CORES * NUM_SUBCORES
B_PER_W = B // NW
CHUNK = 128
N_CHUNKS = B_PER_W // CHUNK


NBUF = 5
LAG = 2
N_GROUPS = N_CHUNKS // NBUF


def _emb_kernel(table_hbm, idx_hbm, out_hbm, idx_v, rows_v, gsems, ssems):
    wid = lax.axis_index("s") * NUM_CORES + lax.axis_index("c")
    base = wid * B_PER_W
    pltpu.sync_copy(idx_hbm.at[pl.ds(base, B_PER_W)], idx_v)

    def gather(j, b):
        return pltpu.make_async_copy(
            table_hbm.at[idx_v.at[pl.ds(j * CHUNK, CHUNK)]],
            rows_v.at[b],
            gsems.at[b],
        )

    def scatter(j, b):
        return pltpu.make_async_copy(
            rows_v.at[b],
            out_hbm.at[pl.ds(base + j * CHUNK, CHUNK)],
            ssems.at[b],
        )

    for b in range(NBUF):
        gather(b, b).start()

    def step(j, b, jl, bl, do_lag):
        gather(j, b).wait()
        scatter(j, b).start()
        if do_lag:
            scatter(jl, bl).wait()
            gather(jl + NBUF, bl).start()

    for b in range(NBUF):
        step(b, b, b - LAG, (b - LAG) % NBUF, b >= LAG)

    def group_body(gi, carry):
        j0 = gi * NBUF
        for b in range(NBUF):
            step(j0 + b, b, j0 + b - LAG, (b - LAG) % NBUF, True)
        return carry

    lax.fori_loop(1, N_GROUPS - 1, group_body, 0)

    j0 = (N_GROUPS - 1) * NBUF
    for b in range(NBUF):
        j = j0 + b
        gather(j, b).wait()
        scatter(j, b).start()
        jl, bl = j - LAG, (b - LAG) % NBUF
        scatter(jl, bl).wait()
        if jl + NBUF < N_CHUNKS:
            gather(jl + NBUF, bl).start()
    for k in range(LAG):
        j = N_CHUNKS - LAG + k
        scatter(j, j % NBUF).wait()


@functools.partial(jax.jit)
def _embedding_lookup(table, idx):
    mesh = plsc.VectorSubcoreMesh(core_axis_name="c", subcore_axis_name="s")
    return pl.kernel(
        _emb_kernel,
        out_type=jax.ShapeDtypeStruct((B, EMBED_DIM), jnp.float32),
        mesh=mesh,
        scratch_types=[
            pltpu.VMEM((B_PER_W,), jnp.int32),
            pltpu.VMEM((NBUF, CHUNK, EMBED_DIM), jnp.float32),
            pltpu.SemaphoreType.DMA((NBUF,)),
            pltpu.SemaphoreType.DMA((NBUF,)),
        ],
    )(table, idx)


def kernel(tokens, token_embedding):
    idx = tokens.reshape(B)
    out = _embedding_lookup(token_embedding, idx)
    return out.reshape(*TOKENS_SHAPE, EMBED_DIM)


# speedup vs baseline: 1.7849x; 1.7849x over previous
import functools

import jax
import jax.numpy as jnp
from jax import lax
from jax.experimental import pallas as pl
from jax.experimental.pallas import tpu as pltpu
from jax.experimental.pallas import tpu_sc as plsc

VOCAB = 100000
EMBED_DIM = 128
ROWS, SEQ = 4096, 50

NUM_CORES = 2
NUM_SUBCORES = 16
NW = NUM_CORES * NUM_SUBCORES
R_PER_W = ROWS // NW

NBUF = 8
LAG = 2
N_GROUPS = R_PER_W // NBUF


def _emb_kernel(table_hbm, tok_hbm, out_hbm, idx_v, rows_v, gsems, ssems):
    wid = lax.axis_index("s") * NUM_CORES + lax.axis_index("c")
    base = wid * R_PER_W
    pltpu.sync_copy(tok_hbm.at[pl.ds(base, R_PER_W)], idx_v)

    def gather(r, b):
        return pltpu.make_async_copy(
            table_hbm.at[idx_v.at[r]],
            rows_v.at[b],
            gsems.at[b],
        )

    def scatter(r, b):
        return pltpu.make_async_copy(
            rows_v.at[b],
            out_hbm.at[base + r],
            ssems.at[b],
        )

    for b in range(NBUF):
        gather(b, b).start()

    def step(r, b, rl, bl, do_lag):
        gather(r, b).wait()
        scatter(r, b).start()
        if do_lag:
            scatter(rl, bl).wait()
            gather(rl + NBUF, bl).start()

    for b in range(NBUF):
        step(b, b, b - LAG, (b - LAG) % NBUF, b >= LAG)

    def group_body(gi, carry):
        r0 = gi * NBUF
        for b in range(NBUF):
            step(r0 + b, b, r0 + b - LAG, (b - LAG) % NBUF, True)
        return carry

    lax.fori_loop(1, N_GROUPS - 1, group_body, 0)

    r0 = (N_GROUPS - 1) * NBUF
    for b in range(NBUF):
        r = r0 + b
        gather(r, b).wait()
        scatter(r, b).start()
        rl, bl = r - LAG, (b - LAG) % NBUF
        scatter(rl, bl).wait()
        if rl + NBUF < R_PER_W:
            gather(rl + NBUF, bl).start()
    for k in range(LAG):
        r = R_PER_W - LAG + k
        scatter(r, r % NBUF).wait()


@functools.partial(jax.jit)
def _embedding_lookup(table, tokens):
    mesh = plsc.VectorSubcoreMesh(core_axis_name="c", subcore_axis_name="s")
    return pl.kernel(
        _emb_kernel,
        out_type=jax.ShapeDtypeStruct((ROWS, SEQ, EMBED_DIM), jnp.float32),
        mesh=mesh,
        scratch_types=[
            pltpu.VMEM((R_PER_W, SEQ), jnp.int32),
            pltpu.VMEM((NBUF, SEQ, EMBED_DIM), jnp.float32),
            pltpu.SemaphoreType.DMA((NBUF,)),
            pltpu.SemaphoreType.DMA((NBUF,)),
        ],
    )(table, tokens)


def kernel(tokens, token_embedding):
    return _embedding_lookup(token_embedding, tokens)


# speedup vs baseline: 1.7872x; 1.0013x over previous
import functools

import jax
import jax.numpy as jnp
from jax import lax
from jax.experimental import pallas as pl
from jax.experimental.pallas import tpu as pltpu
from jax.experimental.pallas import tpu_sc as plsc

VOCAB = 100000
EMBED_DIM = 128
ROWS, SEQ = 4096, 50

NUM_CORES = 2
NUM_SUBCORES = 16
NW = NUM_CORES * NUM_SUBCORES
R_PER_W = ROWS // NW

NBUF = 8
LAG = 2
N_GROUPS = R_PER_W // NBUF


def _emb_kernel(table_hbm, tok_hbm, out_hbm, idx_v, rows_v, gsems, ssems):
    wid = lax.axis_index("s") * NUM_CORES + lax.axis_index("c")
    base = wid * R_PER_W
    pltpu.sync_copy(tok_hbm.at[pl.ds(base, R_PER_W)], idx_v)

    def gather(r, b):
        return pltpu.make_async_copy(
            table_hbm.at[idx_v.at[r]],
            rows_v.at[b],
            gsems.at[b],
        )

    def scatter(r, b):
        return pltpu.make_async_copy(
            rows_v.at[b],
            out_hbm.at[base + r],
            ssems.at[b],
        )

    for b in range(NBUF):
        gather(b, b).start()

    def step(r, b, rl, bl, do_lag):
        gather(r, b).wait()
        scatter(r, b).start()
        if do_lag:
            scatter(rl, bl).wait()
            gather(rl + NBUF, bl).start()

    for b in range(NBUF):
        step(b, b, b - LAG, (b - LAG) % NBUF, b >= LAG)

    def group_body(gi, carry):
        r0 = gi * NBUF
        for b in range(NBUF):
            step(r0 + b, b, r0 + b - LAG, (b - LAG) % NBUF, True)
        return carry

    lax.fori_loop(1, N_GROUPS - 1, group_body, 0)

    r0 = (N_GROUPS - 1) * NBUF
    for b in range(NBUF):
        r = r0 + b
        gather(r, b).wait()
        scatter(r, b).start()
        rl, bl = r - LAG, (b - LAG) % NBUF
        scatter(rl, bl).wait()
        if rl + NBUF < R_PER_W:
            gather(rl + NBUF, bl).start()
    for k in range(LAG):
        r = R_PER_W - LAG + k
        scatter(r, r % NBUF).wait()


@functools.partial(jax.jit)
def _embedding_lookup(table, tokens):
    mesh = plsc.VectorSubcoreMesh(core_axis_name="c", subcore_axis_name="s")
    return pl.kernel(
        _emb_kernel,
        out_type=jax.ShapeDtypeStruct((ROWS, SEQ, EMBED_DIM), jnp.float32),
        mesh=mesh,
        scratch_types=[
            pltpu.VMEM((R_PER_W, SEQ), jnp.int32),
            pltpu.VMEM((NBUF, SEQ, EMBED_DIM), jnp.float32),
            pltpu.SemaphoreType.DMA((NBUF,)),
            pltpu.SemaphoreType.DMA((NBUF,)),
        ],
        compiler_params=pltpu.CompilerParams(use_tc_tiling_on_sc=True),
    )(table, tokens)


def kernel(tokens, token_embedding):
    return _embedding_lookup(token_embedding, tokens)
